# baseline (speedup 1.0000x reference)
import functools

import jax
import jax.numpy as jnp
import numpy as np
from jax.experimental import pallas as pl
from jax.experimental.pallas import tpu as pltpu

_BINS = 10
_EDGES64 = (np.arange(11, dtype=np.float64) / 10.0).astype(np.float32).astype(np.float64)
_THRESH = [float(np.log(e / (1.0 - e))) for e in _EDGES64[1:10]]
_CH = 8
_NACC = 19


def _ghm_body(x_ref, lw_ref, out_ref, acc_ref, *, nsteps, n_total):
    i = pl.program_id(0)
    cols = x_ref.shape[1]
    chunks = x_ref.shape[0] // _CH

    def chunk_fn(k, carry):
        x = x_ref[pl.ds(k * _CH, _CH), :]
        y = jnp.minimum(x * 1.4426950408889634, 126.0)
        bce = 0.6931471805599453 * jnp.log2(1.0 + jnp.exp2(y))
        new = [carry[0] + bce]
        for idx, c in enumerate(_THRESH):
            mf = jnp.where(x >= c, 1.0, 0.0)
            new.append(carry[1 + 2 * idx] + mf)
            new.append(carry[2 + 2 * idx] + mf * bce)
        return tuple(new)

    init = tuple(jnp.zeros((_CH, cols), jnp.float32) for _ in range(_NACC))
    accs = jax.lax.fori_loop(0, chunks, chunk_fn, init, unroll=4)

    @pl.when(i == 0)
    def _init():
        for j in range(_NACC):
            acc_ref[j * _CH:(j + 1) * _CH, :] = accs[j]

    @pl.when(i > 0)
    def _accum():
        for j in range(_NACC):
            sl = slice(j * _CH, (j + 1) * _CH)
            acc_ref[sl, :] += accs[j]

    @pl.when(i == nsteps - 1)
    def _fin():
        lw = lw_ref[0, 0]
        validf = jnp.where(lw > 0.0, jnp.float32(1.0), jnp.float32(0.0))
        zero = jnp.float32(0.0)
        cum_cnt = [jnp.float32(n_total)]
        cum_bce = [jnp.sum(acc_ref[0:_CH, :])]
        for k in range(9):
            cum_cnt.append(jnp.sum(acc_ref[(1 + 2 * k) * _CH:(2 + 2 * k) * _CH, :]))
            cum_bce.append(jnp.sum(acc_ref[(2 + 2 * k) * _CH:(3 + 2 * k) * _CH, :]))
        cum_cnt.append(zero)
        cum_bce.append(zero)
        loss_sum = zero
        n = zero
        for j in range(_BINS):
            cnt = cum_cnt[j] - cum_cnt[j + 1]
            s = cum_bce[j] - cum_bce[j + 1]
            nz = cnt > 0.0
            n += jnp.where(nz, 1.0, 0.0)
            loss_sum += jnp.where(nz, s / jnp.maximum(cnt, 1.0), 0.0)
        loss = jnp.where(n > 0.0, loss_sum / jnp.maximum(n, 1.0), 0.0) * validf
        out_ref[0, 0] = loss


def kernel(pred, target, label_weight):
    n_elems = pred.shape[0] * pred.shape[1]
    if n_elems % 1024 == 0:
        pred = pred.reshape(n_elems // 128, 128)
        target = target.reshape(n_elems // 128, 128)
    x = jnp.where(target > 0, -pred, pred)
    rows, cols = x.shape
    block = 4096
    while rows % block:
        block //= 2
    nsteps = rows // block
    out = pl.pallas_call(
        functools.partial(_ghm_body, nsteps=nsteps, n_total=float(n_elems)),
        grid=(nsteps,),
        in_specs=[
            pl.BlockSpec((block, cols), lambda i: (i, 0)),
            pl.BlockSpec(memory_space=pltpu.SMEM),
        ],
        out_specs=pl.BlockSpec(memory_space=pltpu.SMEM),
        out_shape=jax.ShapeDtypeStruct((1, 1), jnp.float32),
        scratch_shapes=[pltpu.VMEM((_NACC * _CH, cols), jnp.float32)],
        compiler_params=pltpu.CompilerParams(dimension_semantics=("arbitrary",)),
    )(x, label_weight)
    return out[0, 0]


# speedup vs baseline: 3.0807x; 3.0807x over previous
import functools

import jax
import jax.numpy as jnp
import numpy as np
from jax.experimental import pallas as pl
from jax.experimental.pallas import tpu as pltpu

_BINS = 10
_EDGES64 = (np.arange(11, dtype=np.float64) / 10.0).astype(np.float32).astype(np.float64)
_THRESH = [float(np.log(e / (1.0 - e))) for e in _EDGES64[1:10]]
_LANES = 128
_NACC = 19


def _ghm_body(p_ref, t_ref, lw_ref, out_ref, acc_ref, *, steps_total, n_total):
    step = pl.program_id(0) * pl.num_programs(1) + pl.program_id(1)
    chunks = p_ref.shape[1] // _LANES
    rows = p_ref.shape[0]

    def load_x(k):
        p = p_ref[:, pl.ds(k * _LANES, _LANES)]
        t = t_ref[:, pl.ds(k * _LANES, _LANES)]
        return jnp.where(t > 0, -p, p)

    def chunk_fn(k, carry):
        x = load_x(k)
        y = jnp.minimum(x * 1.4426950408889634, 126.0)
        bce = 0.6931471805599453 * jnp.log2(1.0 + jnp.exp2(y))
        new = [carry[0] + bce]
        for idx, c in enumerate(_THRESH):
            mf = jnp.where(x >= c, 1.0, 0.0)
            new.append(carry[1 + 2 * idx] + mf)
            new.append(carry[2 + 2 * idx] + mf * bce)
        return tuple(new)

    zero_acc = jnp.zeros((rows, _LANES), jnp.float32)
    accs = jax.lax.fori_loop(
        0, chunks, chunk_fn, tuple(zero_acc for _ in range(_NACC)), unroll=4)

    @pl.when(step == 0)
    def _init():
        for j in range(_NACC):
            acc_ref[j * rows:(j + 1) * rows, :] = accs[j]

    @pl.when(step > 0)
    def _accum():
        for j in range(_NACC):
            sl = slice(j * rows, (j + 1) * rows)
            acc_ref[sl, :] += accs[j]

    @pl.when(step == steps_total - 1)
    def _fin():
        lw = lw_ref[0, 0]
        validf = jnp.where(lw > 0.0, jnp.float32(1.0), jnp.float32(0.0))
        zero = jnp.float32(0.0)
        cum_cnt = [jnp.float32(n_total)]
        cum_bce = [jnp.sum(acc_ref[0:rows, :])]
        for k in range(9):
            cum_cnt.append(jnp.sum(acc_ref[(1 + 2 * k) * rows:(2 + 2 * k) * rows, :]))
            cum_bce.append(jnp.sum(acc_ref[(2 + 2 * k) * rows:(3 + 2 * k) * rows, :]))
        cum_cnt.append(zero)
        cum_bce.append(zero)
        loss_sum = zero
        n = zero
        for j in range(_BINS):
            cnt = cum_cnt[j] - cum_cnt[j + 1]
            s = cum_bce[j] - cum_bce[j + 1]
            nz = cnt > 0.0
            n += jnp.where(nz, 1.0, 0.0)
            loss_sum += jnp.where(nz, s / jnp.maximum(cnt, 1.0), 0.0)
        loss = jnp.where(n > 0.0, loss_sum / jnp.maximum(n, 1.0), 0.0) * validf
        out_ref[0, 0] = loss


def kernel(pred, target, label_weight):
    n_elems = pred.shape[0] * pred.shape[1]
    pt = pred.T
    tt = target.T.astype(jnp.int32)
    rows, cols = pt.shape
    blk = 32768
    while cols % blk:
        blk //= 2
    row_blk = 8 if rows % 8 == 0 else rows
    g0, g1 = rows // row_blk, cols // blk
    steps_total = g0 * g1
    out = pl.pallas_call(
        functools.partial(
            _ghm_body, steps_total=steps_total, n_total=float(n_elems)),
        grid=(g0, g1),
        in_specs=[
            pl.BlockSpec((row_blk, blk), lambda i, j: (i, j)),
            pl.BlockSpec((row_blk, blk), lambda i, j: (i, j)),
            pl.BlockSpec(memory_space=pltpu.SMEM),
        ],
        out_specs=pl.BlockSpec(memory_space=pltpu.SMEM),
        out_shape=jax.ShapeDtypeStruct((1, 1), jnp.float32),
        scratch_shapes=[pltpu.VMEM((_NACC * row_blk, _LANES), jnp.float32)],
        compiler_params=pltpu.CompilerParams(
            dimension_semantics=("arbitrary", "arbitrary")),
    )(pt, tt, label_weight)
    return out[0, 0]


# speedup vs baseline: 3.3065x; 1.0733x over previous
import functools

import jax
import jax.numpy as jnp
import numpy as np
from jax.experimental import pallas as pl
from jax.experimental.pallas import tpu as pltpu

_BINS = 10
_EDGES64 = (np.arange(11, dtype=np.float64) / 10.0).astype(np.float32).astype(np.float64)
_THRESH = [float(np.log(e / (1.0 - e))) for e in _EDGES64[1:10]]
_LANES = 128
_NACC = 19


def _ghm_body(p_ref, t_ref, lw_ref, out_ref, acc_ref, *, steps_total, n_total):
    step = pl.program_id(0) * pl.num_programs(1) + pl.program_id(1)
    chunks = p_ref.shape[1] // _LANES
    rows = p_ref.shape[0]

    def load_x(k):
        p = p_ref[:, pl.ds(k * _LANES, _LANES)]
        t = t_ref[:, pl.ds(k * _LANES, _LANES)]
        return jnp.where(t > 0, -p, p)

    def chunk_fn(k, carry):
        x = load_x(k)
        y = jnp.minimum(x * 1.4426950408889634, 126.0)
        bce = 0.6931471805599453 * jnp.log2(1.0 + jnp.exp2(y))
        new = [carry[0] + bce]
        for idx, c in enumerate(_THRESH):
            mf = jnp.where(x >= c, 1.0, 0.0)
            new.append(carry[1 + 2 * idx] + mf)
            new.append(carry[2 + 2 * idx] + mf * bce)
        return tuple(new)

    zero_acc = jnp.zeros((rows, _LANES), jnp.float32)
    accs = jax.lax.fori_loop(
        0, chunks, chunk_fn, tuple(zero_acc for _ in range(_NACC)), unroll=6)

    @pl.when(step == 0)
    def _init():
        for j in range(_NACC):
            acc_ref[j * rows:(j + 1) * rows, :] = accs[j]

    @pl.when(step > 0)
    def _accum():
        for j in range(_NACC):
            sl = slice(j * rows, (j + 1) * rows)
            acc_ref[sl, :] += accs[j]

    @pl.when(step == steps_total - 1)
    def _fin():
        lw = lw_ref[0, 0]
        validf = jnp.where(lw > 0.0, jnp.float32(1.0), jnp.float32(0.0))
        zero = jnp.float32(0.0)
        cum_cnt = [jnp.float32(n_total)]
        cum_bce = [jnp.sum(acc_ref[0:rows, :])]
        for k in range(9):
            cum_cnt.append(jnp.sum(acc_ref[(1 + 2 * k) * rows:(2 + 2 * k) * rows, :]))
            cum_bce.append(jnp.sum(acc_ref[(2 + 2 * k) * rows:(3 + 2 * k) * rows, :]))
        cum_cnt.append(zero)
        cum_bce.append(zero)
        loss_sum = zero
        n = zero
        for j in range(_BINS):
            cnt = cum_cnt[j] - cum_cnt[j + 1]
            s = cum_bce[j] - cum_bce[j + 1]
            nz = cnt > 0.0
            n += jnp.where(nz, 1.0, 0.0)
            loss_sum += jnp.where(nz, s / jnp.maximum(cnt, 1.0), 0.0)
        loss = jnp.where(n > 0.0, loss_sum / jnp.maximum(n, 1.0), 0.0) * validf
        out_ref[0, 0] = loss


def kernel(pred, target, label_weight):
    n_elems = pred.shape[0] * pred.shape[1]
    pt = pred.T
    tt = target.T.astype(jnp.int32)
    rows, cols = pt.shape
    blk = 32768
    while cols % blk:
        blk //= 2
    row_blk = 8 if rows % 8 == 0 else rows
    g0, g1 = rows // row_blk, cols // blk
    steps_total = g0 * g1
    out = pl.pallas_call(
        functools.partial(
            _ghm_body, steps_total=steps_total, n_total=float(n_elems)),
        grid=(g0, g1),
        in_specs=[
            pl.BlockSpec((row_blk, blk), lambda i, j: (i, j)),
            pl.BlockSpec((row_blk, blk), lambda i, j: (i, j)),
            pl.BlockSpec(memory_space=pltpu.SMEM),
        ],
        out_specs=pl.BlockSpec(memory_space=pltpu.SMEM),
        out_shape=jax.ShapeDtypeStruct((1, 1), jnp.float32),
        scratch_shapes=[pltpu.VMEM((_NACC * row_blk, _LANES), jnp.float32)],
        compiler_params=pltpu.CompilerParams(
            dimension_semantics=("arbitrary", "arbitrary")),
    )(pt, tt, label_weight)
    return out[0, 0]
